# baseline (speedup 1.0000x reference)
import functools

import jax
import jax.numpy as jnp
from jax import lax
from jax.experimental import pallas as pl
from jax.experimental.pallas import tpu as pltpu
from jax.experimental.pallas import tpu_sc as plsc

NUM_BINS = 8
TB = 3.0
MIN_BIN_WIDTH = 1e-3
MIN_BIN_HEIGHT = 1e-3
MIN_DERIVATIVE = 1e-3

N = 16777216
NW = 32
PER_W = N // NW
CHUNK = 8192
NCH = PER_W // CHUNK
LANES = 16
TBL = 160


def _build_tables(params):
    K = NUM_BINS
    w_raw = params[:K]
    h_raw = params[K:2 * K]
    d_raw = params[2 * K:]
    widths = jax.nn.softmax(w_raw, axis=-1)
    widths = MIN_BIN_WIDTH + (1 - MIN_BIN_WIDTH * K) * widths
    heights = jax.nn.softmax(h_raw, axis=-1)
    heights = MIN_BIN_HEIGHT + (1 - MIN_BIN_HEIGHT * K) * heights
    derivs = MIN_DERIVATIVE + jax.nn.softplus(d_raw)
    widths = 2 * TB * widths
    heights = 2 * TB * heights

    cw = jnp.cumsum(widths)
    cw = jnp.concatenate([jnp.full((1,), -TB, dtype=cw.dtype), cw])
    cw = (cw[:-1] + cw[1:]) / 2
    cw = cw.at[0].set(-TB).at[-1].set(TB)
    ch = jnp.cumsum(heights)
    ch = jnp.concatenate([jnp.full((1,), -TB, dtype=ch.dtype), ch])
    ch = (ch[:-1] + ch[1:]) / 2
    ch = ch.at[0].set(-TB).at[-1].set(TB)

    d = derivs[:K]
    d1 = derivs[1:K + 1]
    delta = heights / widths
    tabs = [
        cw,
        1.0 / widths,
        ch,
        heights * delta,
        heights * d,
        d + d1 - 2 * delta,
        delta,
        delta * delta * d1,
        2 * delta ** 3,
        delta * delta * d,
    ]
    packed = jnp.zeros((10, LANES), jnp.float32)
    for k, t in enumerate(tabs):
        packed = packed.at[k, :K].set(t.astype(jnp.float32))
    return packed.reshape(-1)


def _spline_vec(x, tblv, p4):
    f32 = jnp.float32
    i32 = jnp.int32
    xc = jnp.minimum(jnp.maximum(x, f32(-TB)), f32(TB))

    g1 = (xc >= p4).astype(i32)
    low = 4 * g1
    p2 = plsc.load_gather(tblv, [low + 2])
    low = low + 2 * (xc >= p2).astype(i32)
    p3 = plsc.load_gather(tblv, [low + 1])
    low = low + (xc >= p3).astype(i32)
    pl_ = plsc.load_gather(tblv, [low])
    res = low + (xc >= pl_).astype(i32)
    b = jnp.maximum(res - 1, 0)

    cwb = plsc.load_gather(tblv, [b])
    winv = plsc.load_gather(tblv, [b + 16])
    chb = plsc.load_gather(tblv, [b + 32])
    A = plsc.load_gather(tblv, [b + 48])
    B = plsc.load_gather(tblv, [b + 64])
    C = plsc.load_gather(tblv, [b + 80])
    delta = plsc.load_gather(tblv, [b + 96])
    E = plsc.load_gather(tblv, [b + 112])
    F = plsc.load_gather(tblv, [b + 128])
    G = plsc.load_gather(tblv, [b + 144])

    th = (xc - cwb) * winv
    omt = f32(1.0) - th
    tomt = th * omt
    th2 = th * th
    num = A * th2 + B * tomt
    den = delta + C * tomt
    r = f32(1.0) / den
    out_in = chb + num * r
    dnum = E * th2 + F * tomt + G * (omt * omt)
    larg = (dnum + f32(1e-8)) * r * r

    iv = lax.bitcast_convert_type(larg, i32)
    e = lax.shift_right_logical(iv, 23) - 127
    m = lax.bitcast_convert_type((iv & 0x7FFFFF) | 0x3F800000, f32)
    big = m > f32(1.4142135)
    m = jnp.where(big, m * f32(0.5), m)
    e = jnp.where(big, e + 1, e)
    s = (m - f32(1.0)) / (m + f32(1.0))
    s2 = s * s
    p = ((s2 * f32(1.0 / 7.0) + f32(1.0 / 5.0)) * s2 + f32(1.0 / 3.0)) * s2
    lnm = (f32(2.0) * s) * (f32(1.0) + p)
    lad_in = e.astype(f32) * f32(0.6931471805599453) + lnm
    lad_in = jnp.where(larg == f32(0.0), f32(-jnp.inf), lad_in)

    inside = x == xc
    out = jnp.where(inside, out_in, x)
    lad = jnp.where(inside, lad_in, f32(0.0))
    return out, lad


def _sc_body(x_hbm, tbl_hbm, out_hbm, lad_hbm, tblv, xin, yout, lout):
    wid = lax.axis_index("s") * 2 + lax.axis_index("c")
    base = wid * PER_W
    pltpu.sync_copy(tbl_hbm, tblv)
    p4 = plsc.load_gather(tblv, [jnp.full((LANES,), 4, jnp.int32)])

    def chunk_body(c, carry):
        off = pl.multiple_of(base + c * CHUNK, CHUNK)
        pltpu.sync_copy(x_hbm.at[pl.ds(off, CHUNK)], xin)

        @plsc.parallel_loop(0, CHUNK, step=LANES, unroll=8)
        def vec_body(o):
            x = xin[pl.ds(o, LANES)]
            out, lad = _spline_vec(x, tblv, p4)
            yout[pl.ds(o, LANES)] = out
            lout[pl.ds(o, LANES)] = lad

        pltpu.sync_copy(yout, out_hbm.at[pl.ds(off, CHUNK)])
        pltpu.sync_copy(lout, lad_hbm.at[pl.ds(off, CHUNK)])
        return carry

    lax.fori_loop(0, NCH, chunk_body, 0)


@jax.jit
def kernel(x, params):
    tbl = _build_tables(params)
    mesh = plsc.VectorSubcoreMesh(core_axis_name="c", subcore_axis_name="s")
    f = pl.kernel(
        _sc_body,
        out_type=(
            jax.ShapeDtypeStruct((N,), jnp.float32),
            jax.ShapeDtypeStruct((N,), jnp.float32),
        ),
        mesh=mesh,
        compiler_params=pltpu.CompilerParams(needs_layout_passes=False),
        scratch_types=[
            pltpu.VMEM((TBL,), jnp.float32),
            pltpu.VMEM((CHUNK,), jnp.float32),
            pltpu.VMEM((CHUNK,), jnp.float32),
            pltpu.VMEM((CHUNK,), jnp.float32),
        ],
    )
    return f(x, tbl)


# speedup vs baseline: 1.3794x; 1.3794x over previous
import functools

import jax
import jax.numpy as jnp
from jax import lax
from jax.experimental import pallas as pl
from jax.experimental.pallas import tpu as pltpu
from jax.experimental.pallas import tpu_sc as plsc

NUM_BINS = 8
TB = 3.0
MIN_BIN_WIDTH = 1e-3
MIN_BIN_HEIGHT = 1e-3
MIN_DERIVATIVE = 1e-3

N = 16777216
NW = 32
PER_W = N // NW
CHUNK = 8192
NCH = PER_W // CHUNK
LANES = 16
TBL = 160


def _build_tables(params):
    K = NUM_BINS
    w_raw = params[:K]
    h_raw = params[K:2 * K]
    d_raw = params[2 * K:]
    widths = jax.nn.softmax(w_raw, axis=-1)
    widths = MIN_BIN_WIDTH + (1 - MIN_BIN_WIDTH * K) * widths
    heights = jax.nn.softmax(h_raw, axis=-1)
    heights = MIN_BIN_HEIGHT + (1 - MIN_BIN_HEIGHT * K) * heights
    derivs = MIN_DERIVATIVE + jax.nn.softplus(d_raw)
    widths = 2 * TB * widths
    heights = 2 * TB * heights

    cw = jnp.cumsum(widths)
    cw = jnp.concatenate([jnp.full((1,), -TB, dtype=cw.dtype), cw])
    cw = (cw[:-1] + cw[1:]) / 2
    cw = cw.at[0].set(-TB).at[-1].set(TB)
    ch = jnp.cumsum(heights)
    ch = jnp.concatenate([jnp.full((1,), -TB, dtype=ch.dtype), ch])
    ch = (ch[:-1] + ch[1:]) / 2
    ch = ch.at[0].set(-TB).at[-1].set(TB)

    d = derivs[:K]
    d1 = derivs[1:K + 1]
    delta = heights / widths
    tabs = [
        cw,
        1.0 / widths,
        ch,
        heights * delta,
        heights * d,
        d + d1 - 2 * delta,
        delta,
        delta * delta * d1,
        2 * delta ** 3,
        delta * delta * d,
    ]
    packed = jnp.zeros((10, LANES), jnp.float32)
    for k, t in enumerate(tabs):
        packed = packed.at[k, :K].set(t.astype(jnp.float32))
    return packed.reshape(-1)


def _spline_vec(x, tblv, p4):
    f32 = jnp.float32
    i32 = jnp.int32
    xc = jnp.minimum(jnp.maximum(x, f32(-TB)), f32(TB))

    g1 = (xc >= p4).astype(i32)
    low = 4 * g1
    p2 = plsc.load_gather(tblv, [low + 2])
    low = low + 2 * (xc >= p2).astype(i32)
    p3 = plsc.load_gather(tblv, [low + 1])
    low = low + (xc >= p3).astype(i32)
    pl_ = plsc.load_gather(tblv, [low])
    res = low + (xc >= pl_).astype(i32)
    b = jnp.maximum(res - 1, 0)

    cwb = plsc.load_gather(tblv, [b])
    winv = plsc.load_gather(tblv, [b + 16])
    chb = plsc.load_gather(tblv, [b + 32])
    A = plsc.load_gather(tblv, [b + 48])
    B = plsc.load_gather(tblv, [b + 64])
    C = plsc.load_gather(tblv, [b + 80])
    delta = plsc.load_gather(tblv, [b + 96])
    E = plsc.load_gather(tblv, [b + 112])
    F = plsc.load_gather(tblv, [b + 128])
    G = plsc.load_gather(tblv, [b + 144])

    th = (xc - cwb) * winv
    omt = f32(1.0) - th
    tomt = th * omt
    th2 = th * th
    num = A * th2 + B * tomt
    den = delta + C * tomt
    r = f32(1.0) / den
    out_in = chb + num * r
    dnum = E * th2 + F * tomt + G * (omt * omt)
    larg = (dnum + f32(1e-8)) * r * r

    iv = lax.bitcast_convert_type(larg, i32)
    e = lax.shift_right_logical(iv, 23) - 127
    m = lax.bitcast_convert_type((iv & 0x7FFFFF) | 0x3F800000, f32)
    big = m > f32(1.4142135)
    m = jnp.where(big, m * f32(0.5), m)
    e = jnp.where(big, e + 1, e)
    s = (m - f32(1.0)) / (m + f32(1.0))
    s2 = s * s
    p = ((s2 * f32(1.0 / 7.0) + f32(1.0 / 5.0)) * s2 + f32(1.0 / 3.0)) * s2
    lnm = (f32(2.0) * s) * (f32(1.0) + p)
    lad_in = e.astype(f32) * f32(0.6931471805599453) + lnm
    lad_in = jnp.where(larg == f32(0.0), f32(-jnp.inf), lad_in)

    inside = x == xc
    out = jnp.where(inside, out_in, x)
    lad = jnp.where(inside, lad_in, f32(0.0))
    return out, lad


def _sc_body(x_hbm, tbl_hbm, out_hbm, lad_hbm, tblv, xin, yout, lout,
             sin, sy, sl):
    wid = lax.axis_index("s") * 2 + lax.axis_index("c")
    base = wid * PER_W
    pltpu.sync_copy(tbl_hbm, tblv)
    p4 = plsc.load_gather(tblv, [jnp.full((LANES,), 4, jnp.int32)])

    def in_copy(c, slot):
        off = pl.multiple_of(base + c * CHUNK, CHUNK)
        return pltpu.make_async_copy(
            x_hbm.at[pl.ds(off, CHUNK)], xin.at[slot], sin.at[slot])

    def y_copy(c, slot):
        off = pl.multiple_of(base + c * CHUNK, CHUNK)
        return pltpu.make_async_copy(
            yout.at[slot], out_hbm.at[pl.ds(off, CHUNK)], sy.at[slot])

    def l_copy(c, slot):
        off = pl.multiple_of(base + c * CHUNK, CHUNK)
        return pltpu.make_async_copy(
            lout.at[slot], lad_hbm.at[pl.ds(off, CHUNK)], sl.at[slot])

    def process(c, slot, prefetch, drain):
        in_copy(c, slot).wait()
        @pl.when(drain)
        def _():
            y_copy(c - 2, slot).wait()
            l_copy(c - 2, slot).wait()

        @plsc.parallel_loop(0, CHUNK, step=LANES, unroll=4)
        def vec_body(o):
            x = xin[slot, pl.ds(o, LANES)]
            out, lad = _spline_vec(x, tblv, p4)
            yout[slot, pl.ds(o, LANES)] = out
            lout[slot, pl.ds(o, LANES)] = lad

        y_copy(c, slot).start()
        l_copy(c, slot).start()
        if prefetch:
            in_copy(c + 2, slot).start()

    in_copy(0, 0).start()
    in_copy(1, 1).start()

    def chunk_pair(i, carry):
        c0 = i * 2
        process(c0, 0, True, c0 >= 2)
        process(c0 + 1, 1, True, c0 >= 2)
        return carry

    lax.fori_loop(0, NCH // 2 - 1, chunk_pair, 0)
    process(NCH - 2, 0, False, jnp.bool_(True))
    process(NCH - 1, 1, False, jnp.bool_(True))
    y_copy(NCH - 2, 0).wait()
    l_copy(NCH - 2, 0).wait()
    y_copy(NCH - 1, 1).wait()
    l_copy(NCH - 1, 1).wait()


@jax.jit
def kernel(x, params):
    tbl = _build_tables(params)
    mesh = plsc.VectorSubcoreMesh(core_axis_name="c", subcore_axis_name="s")
    f = pl.kernel(
        _sc_body,
        out_type=(
            jax.ShapeDtypeStruct((N,), jnp.float32),
            jax.ShapeDtypeStruct((N,), jnp.float32),
        ),
        mesh=mesh,
        compiler_params=pltpu.CompilerParams(needs_layout_passes=False),
        scratch_types=[
            pltpu.VMEM((TBL,), jnp.float32),
            pltpu.VMEM((2, CHUNK), jnp.float32),
            pltpu.VMEM((2, CHUNK), jnp.float32),
            pltpu.VMEM((2, CHUNK), jnp.float32),
            pltpu.SemaphoreType.DMA((2,)),
            pltpu.SemaphoreType.DMA((2,)),
            pltpu.SemaphoreType.DMA((2,)),
        ],
    )
    return f(x, tbl)


# speedup vs baseline: 1.4381x; 1.0426x over previous
import functools

import jax
import jax.numpy as jnp
from jax import lax
from jax.experimental import pallas as pl
from jax.experimental.pallas import tpu as pltpu
from jax.experimental.pallas import tpu_sc as plsc

NUM_BINS = 8
TB = 3.0
MIN_BIN_WIDTH = 1e-3
MIN_BIN_HEIGHT = 1e-3
MIN_DERIVATIVE = 1e-3

N = 16777216
NW = 32
PER_W = N // NW
CHUNK = 8192
NCH = PER_W // CHUNK
LANES = 16
TBL = 160


def _build_tables(params):
    K = NUM_BINS
    w_raw = params[:K]
    h_raw = params[K:2 * K]
    d_raw = params[2 * K:]
    widths = jax.nn.softmax(w_raw, axis=-1)
    widths = MIN_BIN_WIDTH + (1 - MIN_BIN_WIDTH * K) * widths
    heights = jax.nn.softmax(h_raw, axis=-1)
    heights = MIN_BIN_HEIGHT + (1 - MIN_BIN_HEIGHT * K) * heights
    derivs = MIN_DERIVATIVE + jax.nn.softplus(d_raw)
    widths = 2 * TB * widths
    heights = 2 * TB * heights

    cw = jnp.cumsum(widths)
    cw = jnp.concatenate([jnp.full((1,), -TB, dtype=cw.dtype), cw])
    cw = (cw[:-1] + cw[1:]) / 2
    cw = cw.at[0].set(-TB).at[-1].set(TB)
    ch = jnp.cumsum(heights)
    ch = jnp.concatenate([jnp.full((1,), -TB, dtype=ch.dtype), ch])
    ch = (ch[:-1] + ch[1:]) / 2
    ch = ch.at[0].set(-TB).at[-1].set(TB)

    d = derivs[:K]
    d1 = derivs[1:K + 1]
    delta = heights / widths
    tabs = [
        cw,
        1.0 / widths,
        ch,
        heights * delta,
        heights * d,
        d + d1 - 2 * delta,
        delta,
        delta * delta * d1,
        2 * delta ** 3,
        delta * delta * d,
    ]
    packed = jnp.zeros((10, LANES), jnp.float32)
    for k, t in enumerate(tabs):
        packed = packed.at[k, :K].set(t.astype(jnp.float32))
    return packed.reshape(-1)


def _spline_vec(x, tblv, p4):
    f32 = jnp.float32
    i32 = jnp.int32
    xc = jnp.minimum(jnp.maximum(x, f32(-TB)), f32(TB))

    g1 = (xc >= p4).astype(i32)
    low = 4 * g1
    p2 = plsc.load_gather(tblv, [low + 2])
    low = low + 2 * (xc >= p2).astype(i32)
    p3 = plsc.load_gather(tblv, [low + 1])
    low = low + (xc >= p3).astype(i32)
    pl_ = plsc.load_gather(tblv, [low])
    res = low + (xc >= pl_).astype(i32)
    b = jnp.maximum(res - 1, 0)

    cwb = plsc.load_gather(tblv, [b])
    winv = plsc.load_gather(tblv, [b + 16])
    chb = plsc.load_gather(tblv, [b + 32])
    A = plsc.load_gather(tblv, [b + 48])
    B = plsc.load_gather(tblv, [b + 64])
    C = plsc.load_gather(tblv, [b + 80])
    delta = plsc.load_gather(tblv, [b + 96])
    E = plsc.load_gather(tblv, [b + 112])
    F = plsc.load_gather(tblv, [b + 128])
    G = plsc.load_gather(tblv, [b + 144])

    th = (xc - cwb) * winv
    omt = f32(1.0) - th
    tomt = th * omt
    th2 = th * th
    num = A * th2 + B * tomt
    den = delta + C * tomt
    r = f32(1.0) / den
    out_in = chb + num * r
    dnum = E * th2 + F * tomt + G * (omt * omt)
    larg = (dnum + f32(1e-8)) * r * r

    iv = lax.bitcast_convert_type(larg, i32)
    e = lax.shift_right_logical(iv, 23) - 127
    m = lax.bitcast_convert_type((iv & 0x7FFFFF) | 0x3F800000, f32)
    t = m - f32(1.0)
    p = f32(0.010243828631049537)
    p = p * t + f32(-0.05326747773316226)
    p = p * t + f32(0.13198966239900722)
    p = p * t + f32(-0.22396689942941594)
    p = p * t + f32(0.3275117137018077)
    p = p * t + f32(-0.49933394898195177)
    p = p * t + f32(0.9999702432977384)
    p = p * t + f32(2.2159764877844275e-07)
    lad_in = e.astype(f32) * f32(0.6931471805599453) + p

    inside = x == xc
    out = jnp.where(inside, out_in, x)
    lad = jnp.where(inside, lad_in, f32(0.0))
    return out, lad


def _sc_body(x_hbm, tbl_hbm, out_hbm, lad_hbm, tblv, xin, yout, lout,
             sin, sy, sl):
    wid = lax.axis_index("s") * 2 + lax.axis_index("c")
    base = wid * PER_W
    pltpu.sync_copy(tbl_hbm, tblv)
    p4 = plsc.load_gather(tblv, [jnp.full((LANES,), 4, jnp.int32)])

    def in_copy(c, slot):
        off = pl.multiple_of(base + c * CHUNK, CHUNK)
        return pltpu.make_async_copy(
            x_hbm.at[pl.ds(off, CHUNK)], xin.at[slot], sin.at[slot])

    def y_copy(c, slot):
        off = pl.multiple_of(base + c * CHUNK, CHUNK)
        return pltpu.make_async_copy(
            yout.at[slot], out_hbm.at[pl.ds(off, CHUNK)], sy.at[slot])

    def l_copy(c, slot):
        off = pl.multiple_of(base + c * CHUNK, CHUNK)
        return pltpu.make_async_copy(
            lout.at[slot], lad_hbm.at[pl.ds(off, CHUNK)], sl.at[slot])

    def process(c, slot, prefetch, drain):
        in_copy(c, slot).wait()
        @pl.when(drain)
        def _():
            y_copy(c - 2, slot).wait()
            l_copy(c - 2, slot).wait()

        @plsc.parallel_loop(0, CHUNK, step=LANES, unroll=4)
        def vec_body(o):
            x = xin[slot, pl.ds(o, LANES)]
            out, lad = _spline_vec(x, tblv, p4)
            yout[slot, pl.ds(o, LANES)] = out
            lout[slot, pl.ds(o, LANES)] = lad

        y_copy(c, slot).start()
        l_copy(c, slot).start()
        if prefetch:
            in_copy(c + 2, slot).start()

    in_copy(0, 0).start()
    in_copy(1, 1).start()

    def chunk_pair(i, carry):
        c0 = i * 2
        process(c0, 0, True, c0 >= 2)
        process(c0 + 1, 1, True, c0 >= 2)
        return carry

    lax.fori_loop(0, NCH // 2 - 1, chunk_pair, 0)
    process(NCH - 2, 0, False, jnp.bool_(True))
    process(NCH - 1, 1, False, jnp.bool_(True))
    y_copy(NCH - 2, 0).wait()
    l_copy(NCH - 2, 0).wait()
    y_copy(NCH - 1, 1).wait()
    l_copy(NCH - 1, 1).wait()


@jax.jit
def kernel(x, params):
    tbl = _build_tables(params)
    mesh = plsc.VectorSubcoreMesh(core_axis_name="c", subcore_axis_name="s")
    f = pl.kernel(
        _sc_body,
        out_type=(
            jax.ShapeDtypeStruct((N,), jnp.float32),
            jax.ShapeDtypeStruct((N,), jnp.float32),
        ),
        mesh=mesh,
        compiler_params=pltpu.CompilerParams(needs_layout_passes=False),
        scratch_types=[
            pltpu.VMEM((TBL,), jnp.float32),
            pltpu.VMEM((2, CHUNK), jnp.float32),
            pltpu.VMEM((2, CHUNK), jnp.float32),
            pltpu.VMEM((2, CHUNK), jnp.float32),
            pltpu.SemaphoreType.DMA((2,)),
            pltpu.SemaphoreType.DMA((2,)),
            pltpu.SemaphoreType.DMA((2,)),
        ],
    )
    return f(x, tbl)
